# baseline (speedup 1.0000x reference)
import functools

import jax
import jax.numpy as jnp
from jax import lax
from jax.experimental import pallas as pl
from jax.experimental.pallas import tpu as pltpu
from jax.experimental.pallas import tpu_sc as plsc

_NC, _NS, _LANES = 2, 16, 16
_NW = _NC * _NS
_D = 256
_W = 2 * _D
_P = 64 * 1024
_PPW = _P // _NW
_CH = 64
_NCH = _PPW // _CH
_IDXROWS = _PPW // _CH
_XPT = 64 // _NS


@functools.partial(
    pl.kernel,
    mesh=plsc.VectorSubcoreMesh(core_axis_name="c", subcore_axis_name="s"),
    out_type=(
        jax.ShapeDtypeStruct((_P, _W), jnp.float32),
        jax.ShapeDtypeStruct((64 * 64, _W), jnp.float32),
        jax.ShapeDtypeStruct((64 * 64, _W), jnp.float32),
    ),
    scratch_types=[
        pltpu.VMEM((_IDXROWS, _CH), jnp.int32),
        pltpu.VMEM((_IDXROWS, _CH), jnp.int32),
        pltpu.VMEM((_XPT, _D), jnp.float32),
        pltpu.VMEM((64, _D), jnp.float32),
        pltpu.VMEM((64, _W), jnp.float32),
        pltpu.VMEM((_CH, _W), jnp.float32),
        pltpu.VMEM((_CH, _W), jnp.float32),
        pltpu.SemaphoreType.DMA,
        pltpu.SemaphoreType.DMA,
        pltpu.SemaphoreType.DMA,
        pltpu.SemaphoreType.DMA,
        pltpu.SemaphoreType.DMA,
        pltpu.SemaphoreType.DMA,
    ],
)
def _sc_lookup(idx_x_hbm, idx_y_hbm, col_hbm, row_hbm,
               out_hbm, cross0_hbm, cross1_hbm,
               idxx_v, idxy_v, colblk_v, rowtbl_v, blk_v, buf0, buf1,
               sg0, sg1, sg2, so0, so1, so2):
    bufs = (buf0, buf1, blk_v)
    sgs = (sg0, sg1, sg2)
    sos = (so0, so1, so2)
    cid = lax.axis_index("c")
    sid = lax.axis_index("s")
    wid = sid * _NC + cid
    base = wid * _PPW

    pltpu.sync_copy(col_hbm.at[pl.ds(sid * _XPT, _XPT)], colblk_v)
    pltpu.sync_copy(row_hbm, rowtbl_v)

    def build_into(cross_hbm):
        for xi in range(_XPT):
            left = [colblk_v[xi, pl.ds(k * _LANES, _LANES)]
                    for k in range(_D // _LANES)]

            def fill_row(r, carry):
                for k in range(_D // _LANES):
                    blk_v[r, pl.ds(k * _LANES, _LANES)] = left[k]
                for k in range(_D // _LANES):
                    blk_v[r, pl.ds(_D + k * _LANES, _LANES)] = (
                        rowtbl_v[r, pl.ds(k * _LANES, _LANES)])
                return carry

            lax.fori_loop(0, 64, fill_row, 0)
            pltpu.sync_copy(
                blk_v, cross_hbm.at[pl.ds((sid * _XPT + xi) * 64, 64)])

    @pl.when(cid == 0)
    def _b0():
        build_into(cross0_hbm)

    @pl.when(cid == 1)
    def _b1():
        build_into(cross1_hbm)

    pltpu.sync_copy(idx_x_hbm.at[pl.ds(wid * _IDXROWS, _IDXROWS)], idxx_v)
    pltpu.sync_copy(idx_y_hbm.at[pl.ds(wid * _IDXROWS, _IDXROWS)], idxy_v)

    def comb_row(i, carry):
        def comb_vec(j, c2):
            sl = pl.ds(j * _LANES, _LANES)
            idxx_v[i, sl] = idxx_v[i, sl] * 64 + idxy_v[i, sl]
            return c2
        return lax.fori_loop(0, _CH // _LANES, comb_vec, carry)

    lax.fori_loop(0, _IDXROWS, comb_row, 0)

    plsc.subcore_barrier()

    def gather_phase(cross_hbm):
        def start_gather(c, b):
            pltpu.async_copy(cross_hbm.at[idxx_v.at[c]], bufs[b], sgs[b])

        def wait_gather(b):
            pltpu.make_async_copy(
                cross_hbm.at[idxx_v.at[0]], bufs[b], sgs[b]).wait()

        def out_desc(c, b):
            return pltpu.make_async_copy(
                bufs[b], out_hbm.at[pl.ds(base + c * _CH, _CH)], sos[b])

        start_gather(0, 0)
        start_gather(1, 1)

        wait_gather(0)
        out_desc(0, 0).start()
        start_gather(2, 2)
        wait_gather(1)
        out_desc(1, 1).start()
        out_desc(0, 0).wait()
        start_gather(3, 0)
        wait_gather(2)
        out_desc(2, 2).start()
        out_desc(1, 1).wait()
        start_gather(4, 1)

        def trip(s, carry):
            for b in range(3):
                c = 3 * s + b
                wait_gather(b)
                out_desc(c, b).start()
                bp = (b + 2) % 3
                out_desc(c - 1, bp).wait()
                start_gather(c + 2, bp)
            return carry

        lax.fori_loop(1, _NCH // 3, trip, 0)

        wait_gather(0)
        out_desc(30, 0).start()
        out_desc(29, 2).wait()
        wait_gather(1)
        out_desc(31, 1).start()
        out_desc(30, 0).wait()
        out_desc(31, 1).wait()

    @pl.when(cid == 0)
    def _g0():
        gather_phase(cross0_hbm)

    @pl.when(cid == 1)
    def _g1():
        gather_phase(cross1_hbm)


def kernel(position_inds, col_embed, row_embed):
    pi = position_inds.astype(jnp.int32)
    idx_x = pi[:, :, 0].reshape(_P // _CH, _CH)
    idx_y = pi[:, :, 1].reshape(_P // _CH, _CH)
    out, _, _ = _sc_lookup(idx_x, idx_y, col_embed, row_embed)
    return out.reshape(64, 1024, _W)
